# Initial kernel scaffold; baseline (speedup 1.0000x reference)
import jax
import jax.numpy as jnp
from jax.experimental import pallas as pl


def kernel(x, Wg, W, b):
    raise NotImplementedError("write your pallas kernel here")


# speedup vs baseline: 1.9526x; 1.9526x over previous
import functools

import jax
import jax.numpy as jnp
from jax.experimental import pallas as pl
from jax.experimental.pallas import tpu as pltpu

DM = 768
NE = 8
NT = 2048
BT = 256


def _moe_body(x_ref, wg_ref, w_ref, b_ref, o_ref):
    x = x_ref[...]
    gate = jax.lax.dot_general(
        x, wg_ref[...], (((1,), (0,)), ((), ())),
        preferred_element_type=jnp.float32)
    iota = jax.lax.broadcasted_iota(jnp.int32, gate.shape, 1)
    v1 = jnp.max(gate, axis=1, keepdims=True)
    i1 = jnp.min(jnp.where(gate >= v1, iota, NE), axis=1, keepdims=True)
    g2 = jnp.where(iota == i1, -jnp.inf, gate)
    v2 = jnp.max(g2, axis=1, keepdims=True)
    i2 = jnp.min(jnp.where(g2 >= v2, iota, NE), axis=1, keepdims=True)
    p1 = 1.0 / (1.0 + jnp.exp(v2 - v1))
    estar = jnp.maximum(i1, i2)
    wstar = jnp.where(i1 >= i2, p1, 1.0 - p1)
    out = jnp.zeros_like(x)
    for e in range(NE):
        eo = jax.lax.dot_general(
            x, w_ref[e], (((1,), (1,)), ((), ())),
            preferred_element_type=jnp.float32) + b_ref[e][None, :]
        out = jnp.where(estar == e, wstar * eo, out)
    o_ref[...] = out


def kernel(x, Wg, W, b):
    return pl.pallas_call(
        _moe_body,
        grid=(NT // BT,),
        in_specs=[
            pl.BlockSpec((BT, DM), lambda i: (i, 0)),
            pl.BlockSpec((DM, NE), lambda i: (0, 0)),
            pl.BlockSpec((NE, DM, DM), lambda i: (0, 0, 0)),
            pl.BlockSpec((NE, DM), lambda i: (0, 0)),
        ],
        out_specs=pl.BlockSpec((BT, DM), lambda i: (i, 0)),
        out_shape=jax.ShapeDtypeStruct((NT, DM), jnp.float32),
    )(x, Wg, W, b)
